# baseline (speedup 1.0000x reference)
import functools

import jax
import jax.numpy as jnp
from jax import lax
from jax.experimental import pallas as pl
from jax.experimental.pallas import tpu as pltpu
from jax.experimental.pallas import tpu_sc as plsc

DIM = 768
HID = 2048
E = 16
TPAD = 128
ROWS = 16


def _tc_body(order_ref, len_ref, x_ref, w13_ref, w2_ref, out_ref):
    e = pl.program_id(0)
    eid = order_ref[e]
    prev = order_ref[jnp.maximum(e - 1, 0)]
    is_rep = jnp.logical_and(e > 0, eid == prev)

    def acc(i, s):
        return s + jnp.where(i < eid, len_ref[i], 0)
    start = lax.fori_loop(0, E, acc, 0)
    cnt = jnp.where(is_rep, 0, len_ref[eid])

    ri = lax.broadcasted_iota(jnp.int32, (ROWS, TPAD), 0)
    ti = lax.broadcasted_iota(jnp.int32, (ROWS, TPAD), 1)
    sel = jnp.logical_and(ti == start + ri, ri < cnt)
    p = sel.astype(jnp.float32)

    xe = jnp.dot(p, x_ref[...], preferred_element_type=jnp.float32)
    inter = jnp.dot(xe, w13_ref[0], preferred_element_type=jnp.float32)
    x1 = inter[:, :HID]
    x3 = inter[:, HID:]
    h = x1 * jax.nn.sigmoid(x1) * x3
    oe = jnp.dot(h, w2_ref[0], preferred_element_type=jnp.float32)

    @pl.when(e == 0)
    def _():
        out_ref[...] = jnp.zeros_like(out_ref)
    out_ref[...] += jnp.dot(p.T, oe, preferred_element_type=jnp.float32)


def _sc_body(comb_hbm, out_hbm, comb_v, avg_v, tf_v):
    c = lax.axis_index("c")
    s = lax.axis_index("s")

    @pl.when(jnp.logical_and(c == 0, s == 0))
    def _():
        pltpu.sync_copy(comb_hbm, comb_v)

        lanes = lax.iota(jnp.int32, 16)

        def cumsum16(vec):
            for shift in (1, 2, 4, 8):
                tf_v[...] = vec
                g = plsc.load_gather(tf_v, [jnp.maximum(lanes - shift, 0)])
                vec = vec + jnp.where(lanes >= shift, g, 0.0)
            return vec

        carry = jnp.zeros((16,), jnp.float32)
        for k in range(TPAD // 16):
            cs = cumsum16(comb_v[pl.ds(k * 16, 16)]) + carry
            comb_v[pl.ds(k * 16, 16)] = cs
            tf_v[...] = cs
            carry = plsc.load_gather(tf_v, [jnp.full((16,), 15, jnp.int32)])

        lv = comb_v[pl.ds(TPAD, 16)]
        cum = cumsum16(lv)
        idx_end = cum.astype(jnp.int32) - 1
        idx_start = (cum - lv).astype(jnp.int32) - 1
        ge = plsc.load_gather(comb_v, [jnp.maximum(idx_end, 0)])
        gs = plsc.load_gather(comb_v, [jnp.maximum(idx_start, 0)])
        ge = jnp.where(idx_end >= 0, ge, 0.0)
        gs = jnp.where(idx_start >= 0, gs, 0.0)
        avg_v[...] = (ge - gs) / jnp.maximum(lv, 1.0)
        pltpu.sync_copy(avg_v, out_hbm)


_sc_avg = functools.partial(
    pl.kernel,
    out_type=jax.ShapeDtypeStruct((E,), jnp.float32),
    mesh=plsc.VectorSubcoreMesh(core_axis_name="c", subcore_axis_name="s"),
    compiler_params=pltpu.CompilerParams(needs_layout_passes=False),
    scratch_types=[
        pltpu.VMEM((TPAD + E,), jnp.float32),
        pltpu.VMEM((E,), jnp.float32),
        pltpu.VMEM((16,), jnp.float32),
    ],
)(_sc_body)


@jax.jit
def kernel(x, num_tokens_per_expert, top_scores, w13, w2):
    T = x.shape[0]
    lengths = num_tokens_per_expert.astype(jnp.int32)
    xp = jnp.zeros((TPAD, DIM), jnp.float32).at[:T].set(x)
    comb = jnp.zeros((TPAD + E,), jnp.float32)
    comb = comb.at[:T].set(top_scores).at[TPAD:].set(lengths.astype(jnp.float32))

    avg = _sc_avg(comb)

    act = lengths > 0
    key = jnp.where(act, jnp.arange(E, dtype=jnp.int32), E)
    orders = jnp.sort(key)
    n_act = jnp.sum(act.astype(jnp.int32))
    last = jnp.where(n_act > 0, orders[jnp.maximum(n_act - 1, 0)], 0)
    order = jnp.where(jnp.arange(E) < n_act, orders, last).astype(jnp.int32)

    out_p = pl.pallas_call(
        _tc_body,
        grid_spec=pltpu.PrefetchScalarGridSpec(
            num_scalar_prefetch=2,
            grid=(E,),
            in_specs=[
                pl.BlockSpec((TPAD, DIM), lambda e, o, l: (0, 0)),
                pl.BlockSpec((1, DIM, 2 * HID), lambda e, o, l: (o[e], 0, 0)),
                pl.BlockSpec((1, HID, DIM), lambda e, o, l: (o[e], 0, 0)),
            ],
            out_specs=pl.BlockSpec((TPAD, DIM), lambda e, o, l: (0, 0)),
        ),
        out_shape=jax.ShapeDtypeStruct((TPAD, DIM), jnp.float32),
        compiler_params=pltpu.CompilerParams(
            dimension_semantics=("arbitrary",),
        ),
    )(order, lengths, xp, w13, w2)

    return out_p[:T], avg


# speedup vs baseline: 1.1522x; 1.1522x over previous
import functools

import jax
import jax.numpy as jnp
from jax import lax
from jax.experimental import pallas as pl
from jax.experimental.pallas import tpu as pltpu
from jax.experimental.pallas import tpu_sc as plsc

DIM = 768
HID = 2048
E = 16
TPAD = 128
ROWS = 16


def _tc_body(order_ref, len_ref, x_ref, scores_ref, w13_ref, w2_ref, out_ref, avg_ref):
    e = pl.program_id(0)
    eid = order_ref[e]
    prev = order_ref[jnp.maximum(e - 1, 0)]
    is_rep = jnp.logical_and(e > 0, eid == prev)

    def acc(i, s):
        return s + jnp.where(i < eid, len_ref[i], 0)
    start = lax.fori_loop(0, E, acc, 0)
    cnt = jnp.where(is_rep, 0, len_ref[eid])

    ri = lax.broadcasted_iota(jnp.int32, (ROWS, TPAD), 0)
    ti = lax.broadcasted_iota(jnp.int32, (ROWS, TPAD), 1)
    sel = jnp.logical_and(ti == start + ri, ri < cnt)
    p = sel.astype(jnp.float32)

    xe = jnp.dot(p, x_ref[...], preferred_element_type=jnp.float32)
    inter = jnp.dot(xe, w13_ref[0], preferred_element_type=jnp.float32)
    x1 = inter[:, :HID]
    x3 = inter[:, HID:]
    h = x1 * jax.nn.sigmoid(x1) * x3
    oe = jnp.dot(h, w2_ref[0], preferred_element_type=jnp.float32)

    @pl.when(e == 0)
    def _():
        out_ref[...] = jnp.zeros_like(out_ref)
        avg_ref[...] = jnp.zeros_like(avg_ref)
    out_ref[...] += jnp.dot(p.T, oe, preferred_element_type=jnp.float32)

    col = lax.broadcasted_iota(jnp.int32, (1, TPAD), 1)
    in_seg = jnp.logical_and(col >= start, col < start + cnt)
    ssum = jnp.sum(jnp.where(in_seg, scores_ref[...], 0.0))
    avg = ssum / jnp.maximum(cnt, 1).astype(jnp.float32)
    lane = lax.broadcasted_iota(jnp.int32, (1, E), 1)
    wr = jnp.logical_and(lane == eid, jnp.logical_not(is_rep))
    avg_ref[...] = jnp.where(wr, avg, avg_ref[...])


def _sc_body(comb_hbm, out_hbm, comb_v, avg_v, tf_v):
    c = lax.axis_index("c")
    s = lax.axis_index("s")

    @pl.when(jnp.logical_and(c == 0, s == 0))
    def _():
        pltpu.sync_copy(comb_hbm, comb_v)

        lanes = lax.iota(jnp.int32, 16)

        def cumsum16(vec):
            for shift in (1, 2, 4, 8):
                tf_v[...] = vec
                g = plsc.load_gather(tf_v, [jnp.maximum(lanes - shift, 0)])
                vec = vec + jnp.where(lanes >= shift, g, 0.0)
            return vec

        carry = jnp.zeros((16,), jnp.float32)
        for k in range(TPAD // 16):
            cs = cumsum16(comb_v[pl.ds(k * 16, 16)]) + carry
            comb_v[pl.ds(k * 16, 16)] = cs
            tf_v[...] = cs
            carry = plsc.load_gather(tf_v, [jnp.full((16,), 15, jnp.int32)])

        lv = comb_v[pl.ds(TPAD, 16)]
        cum = cumsum16(lv)
        idx_end = cum.astype(jnp.int32) - 1
        idx_start = (cum - lv).astype(jnp.int32) - 1
        ge = plsc.load_gather(comb_v, [jnp.maximum(idx_end, 0)])
        gs = plsc.load_gather(comb_v, [jnp.maximum(idx_start, 0)])
        ge = jnp.where(idx_end >= 0, ge, 0.0)
        gs = jnp.where(idx_start >= 0, gs, 0.0)
        avg_v[...] = (ge - gs) / jnp.maximum(lv, 1.0)
        pltpu.sync_copy(avg_v, out_hbm)


_sc_avg = functools.partial(
    pl.kernel,
    out_type=jax.ShapeDtypeStruct((E,), jnp.float32),
    mesh=plsc.VectorSubcoreMesh(core_axis_name="c", subcore_axis_name="s"),
    compiler_params=pltpu.CompilerParams(needs_layout_passes=False),
    scratch_types=[
        pltpu.VMEM((TPAD + E,), jnp.float32),
        pltpu.VMEM((E,), jnp.float32),
        pltpu.VMEM((16,), jnp.float32),
    ],
)(_sc_body)


@jax.jit
def kernel(x, num_tokens_per_expert, top_scores, w13, w2):
    T = x.shape[0]
    lengths = num_tokens_per_expert.astype(jnp.int32)
    xp = jnp.zeros((TPAD, DIM), jnp.float32).at[:T].set(x)
    comb = jnp.zeros((TPAD + E,), jnp.float32)
    comb = comb.at[:T].set(top_scores).at[TPAD:].set(lengths.astype(jnp.float32))

    sp = jnp.zeros((1, TPAD), jnp.float32).at[0, :T].set(top_scores)

    act = lengths > 0
    key = jnp.where(act, jnp.arange(E, dtype=jnp.int32), E)
    orders = jnp.sort(key)
    n_act = jnp.sum(act.astype(jnp.int32))
    last = jnp.where(n_act > 0, orders[jnp.maximum(n_act - 1, 0)], 0)
    order = jnp.where(jnp.arange(E) < n_act, orders, last).astype(jnp.int32)

    out_p, avg = pl.pallas_call(
        _tc_body,
        grid_spec=pltpu.PrefetchScalarGridSpec(
            num_scalar_prefetch=2,
            grid=(E,),
            in_specs=[
                pl.BlockSpec((TPAD, DIM), lambda e, o, l: (0, 0)),
                pl.BlockSpec((1, TPAD), lambda e, o, l: (0, 0)),
                pl.BlockSpec((1, DIM, 2 * HID), lambda e, o, l: (o[e], 0, 0)),
                pl.BlockSpec((1, HID, DIM), lambda e, o, l: (o[e], 0, 0)),
            ],
            out_specs=[
                pl.BlockSpec((TPAD, DIM), lambda e, o, l: (0, 0)),
                pl.BlockSpec((1, E), lambda e, o, l: (0, 0)),
            ],
        ),
        out_shape=[
            jax.ShapeDtypeStruct((TPAD, DIM), jnp.float32),
            jax.ShapeDtypeStruct((1, E), jnp.float32),
        ],
        compiler_params=pltpu.CompilerParams(
            dimension_semantics=("arbitrary",),
        ),
    )(order, lengths, xp, sp, w13, w2)

    return out_p[:T], avg[0]
